# baseline (speedup 1.0000x reference)
import functools

import jax
import jax.numpy as jnp
from jax import lax
from jax.experimental import pallas as pl
from jax.experimental.pallas import tpu as pltpu
from jax.experimental.pallas import tpu_sc as plsc

_B, _N, _F = 4, 4096, 128
_K = 2048
_IT_A = 512
_IT_R = 256
_NW = 32
_RPW = _K // _NW
_RB = 8
_L = 16
_NB = 1


def _scores_body(x_ref, kn_ref, y_ref, xg_ref):
    kn = kn_ref[...]
    sq = jnp.sum(kn * kn)
    knn = kn * lax.rsqrt(jnp.maximum(sq, 1e-12))
    x = x_ref[0]
    y = lax.dot_general(
        x.astype(jnp.bfloat16), knn.astype(jnp.bfloat16),
        (((1,), (0,)), ((), ())),
        preferred_element_type=jnp.float32)
    y_ref[0] = y
    xg_ref[0] = x * jnp.tanh(y)


def _rank_body(yi_ref, yj_ref, r_ref):
    yi = yi_ref[0]
    yj = yj_ref[0]
    i0 = pl.program_id(1) * _IT_R
    ii = i0 + lax.broadcasted_iota(jnp.int32, (_IT_R, _N), 0)
    jj = lax.broadcasted_iota(jnp.int32, (_IT_R, _N), 1)
    one, zero = jnp.int32(1), jnp.int32(0)
    ge = jnp.where(yj >= yi, one, zero)
    gt = jnp.where(yj > yi, one, zero)
    cnt = jnp.where(jj < ii, ge, gt)
    r_ref[0] = jnp.sum(cnt, axis=1, keepdims=True)


def _sc_body(xg_hbm, a_hbm, rank_hbm, newx_hbm, newa_hbm, idx_hbm,
             rank_v, idx_v, xrows_v, arows0_v, arows1_v, aout0_v, aout1_v,
             sem_x, sem_a0, sem_a1, sem_o0, sem_o1):
    wid = lax.axis_index("s") * 2 + lax.axis_index("c")
    base = wid * _RPW
    lanes = lax.iota(jnp.int32, _L)
    bufs = (arows0_v, arows1_v)
    sems = (sem_a0, sem_a1)
    outs = (aout0_v, aout1_v)
    osems = (sem_o0, sem_o1)

    for b in range(_NB):
        pltpu.sync_copy(rank_hbm.at[b], rank_v)

        @plsc.parallel_loop(0, _N // _L, unroll=4)
        def _(j):
            rv = rank_v[pl.ds(j * _L, _L)]
            iv = j * _L + lanes
            plsc.store_scatter(idx_v, [rv], iv, mask=rv < _K)

        pltpu.sync_copy(idx_v.at[pl.ds(base, _RPW)],
                        idx_hbm.at[b].at[pl.ds(base, _RPW)])

        pltpu.async_copy(xg_hbm.at[b].at[idx_v.at[pl.ds(base, _RPW)]],
                         xrows_v, sem_x).wait()
        pltpu.sync_copy(xrows_v, newx_hbm.at[b].at[pl.ds(base, _RPW)])

        nblk = _RPW // _RB
        descs = [None, None]
        odescs = [None, None]
        descs[0] = pltpu.async_copy(
            a_hbm.at[b].at[idx_v.at[pl.ds(base, _RB)]], bufs[0], sems[0])
        for t in range(nblk):
            if t + 1 < nblk:
                descs[(t + 1) % 2] = pltpu.async_copy(
                    a_hbm.at[b].at[idx_v.at[pl.ds(base + (t + 1) * _RB, _RB)]],
                    bufs[(t + 1) % 2], sems[(t + 1) % 2])
            descs[t % 2].wait()
            if odescs[t % 2] is not None:
                odescs[t % 2].wait()
            rows = bufs[t % 2]
            aout = outs[t % 2]

            @plsc.parallel_loop(0, _K // _L, unroll=2)
            def _(j, rows=rows, aout=aout):
                cv = idx_v[pl.ds(j * _L, _L)]
                for r in range(_RB):
                    rv = jnp.full((_L,), r, jnp.int32)
                    aout[r, pl.ds(j * _L, _L)] = plsc.load_gather(
                        rows, [rv, cv])

            odescs[t % 2] = pltpu.async_copy(
                aout, newa_hbm.at[b].at[pl.ds(base + t * _RB, _RB)],
                osems[t % 2])
        for dsc in odescs:
            if dsc is not None:
                dsc.wait()


@functools.cache
def _sc_gather():
    return pl.kernel(
        _sc_body,
        out_type=[
            jax.ShapeDtypeStruct((_NB, _K, _F), jnp.float32),
            jax.ShapeDtypeStruct((_NB, _K, _K), jnp.float32),
            jax.ShapeDtypeStruct((_NB, _K), jnp.int32),
        ],
        mesh=plsc.VectorSubcoreMesh(core_axis_name="c", subcore_axis_name="s"),
        compiler_params=pltpu.CompilerParams(needs_layout_passes=False),
        scratch_types=[
            pltpu.VMEM((_N,), jnp.int32),
            pltpu.VMEM((_K,), jnp.int32),
            pltpu.VMEM((_RPW, _F), jnp.float32),
            pltpu.VMEM((_RB, _N), jnp.float32),
            pltpu.VMEM((_RB, _N), jnp.float32),
            pltpu.VMEM((_RB, _K), jnp.float32),
            pltpu.VMEM((_RB, _K), jnp.float32),
            pltpu.SemaphoreType.DMA,
            pltpu.SemaphoreType.DMA,
            pltpu.SemaphoreType.DMA,
            pltpu.SemaphoreType.DMA,
            pltpu.SemaphoreType.DMA,
        ],
    )


def _tc_stage(Xb, kn):
    y3, xg = pl.pallas_call(
        _scores_body,
        grid=(_NB, _N // _IT_A),
        in_specs=[
            pl.BlockSpec((1, _IT_A, _F), lambda b, i: (b, i, 0)),
            pl.BlockSpec((_F, 1), lambda b, i: (0, 0)),
        ],
        out_specs=[
            pl.BlockSpec((1, _IT_A, 1), lambda b, i: (b, i, 0)),
            pl.BlockSpec((1, _IT_A, _F), lambda b, i: (b, i, 0)),
        ],
        out_shape=[
            jax.ShapeDtypeStruct((_NB, _N, 1), jnp.float32),
            jax.ShapeDtypeStruct((_NB, _N, _F), jnp.float32),
        ],
    )(Xb, kn)

    y_row = y3.reshape(_NB, 1, _N)
    rank3 = pl.pallas_call(
        _rank_body,
        grid=(_NB, _N // _IT_R),
        in_specs=[
            pl.BlockSpec((1, _IT_R, 1), lambda b, i: (b, i, 0)),
            pl.BlockSpec((1, 1, _N), lambda b, i: (b, 0, 0)),
        ],
        out_specs=pl.BlockSpec((1, _IT_R, 1), lambda b, i: (b, i, 0)),
        out_shape=jax.ShapeDtypeStruct((_NB, _N, 1), jnp.int32),
    )(y3, y_row)
    return xg, rank3.reshape(_NB, _N)


def kernel(X, A, kernel, training):
    nxs, nas, ixs = [], [], []
    for b0 in range(0, _B, _NB):
        xg, rank = _tc_stage(X[b0:b0 + _NB], kernel)
        nx, na, ix = _sc_gather()(xg, A[b0:b0 + _NB], rank)
        nxs.append(nx)
        nas.append(na)
        ixs.append(ix)
    new_x = jnp.concatenate(nxs, axis=0) if len(nxs) > 1 else nxs[0]
    new_a = jnp.concatenate(nas, axis=0) if len(nas) > 1 else nas[0]
    idx = jnp.concatenate(ixs, axis=0) if len(ixs) > 1 else ixs[0]
    return (new_x, new_a, idx[..., None])


# speedup vs baseline: 2.0490x; 2.0490x over previous
import functools

import jax
import jax.numpy as jnp
from jax import lax
from jax.experimental import pallas as pl
from jax.experimental.pallas import tpu as pltpu
from jax.experimental.pallas import tpu_sc as plsc

_B, _N, _F = 4, 4096, 128
_K = 2048
_IT_A = 2048
_IT_R = 512
_NW = 32
_RPW = _K // _NW
_RB = 8
_L = 16


def _scores_body(x_ref, kn_ref, y_ref, xg_ref):
    kn = kn_ref[...]
    sq = jnp.sum(kn * kn)
    knn = kn * lax.rsqrt(jnp.maximum(sq, 1e-12))
    x = x_ref[0]
    y = lax.dot_general(
        x.astype(jnp.bfloat16), knn.astype(jnp.bfloat16),
        (((1,), (0,)), ((), ())),
        preferred_element_type=jnp.float32)
    y_ref[0] = y
    xg_ref[0] = x * jnp.tanh(y)


def _rank_body(yi_ref, yj_ref, r_ref):
    yi = yi_ref[0]
    yj = yj_ref[0]
    i0 = pl.program_id(1) * _IT_R
    ii = i0 + lax.broadcasted_iota(jnp.int32, (_IT_R, _N), 0)
    jj = lax.broadcasted_iota(jnp.int32, (_IT_R, _N), 1)
    one, zero = jnp.float32(1), jnp.float32(0)
    ge = jnp.where(yj >= yi, one, zero)
    gt = jnp.where(yj > yi, one, zero)
    cnt = jnp.where(jj < ii, ge, gt)
    ones = jnp.ones((_N, 1), jnp.float32)
    s = lax.dot_general(cnt, ones, (((1,), (0,)), ((), ())),
                        preferred_element_type=jnp.float32)
    r_ref[0] = s.astype(jnp.int32)


def _sc_body(xg_hbm, a_hbm, rank_hbm, newx_hbm, newa_hbm, idx_hbm,
             rank_v, idx_v, xrows_v, arows0_v, arows1_v, aout0_v, aout1_v,
             sem_x, sem_a0, sem_a1, sem_o0, sem_o1):
    wid = lax.axis_index("s") * 2 + lax.axis_index("c")
    base = wid * _RPW
    lanes = lax.iota(jnp.int32, _L)
    bufs = (arows0_v, arows1_v)
    sems = (sem_a0, sem_a1)
    outs = (aout0_v, aout1_v)
    osems = (sem_o0, sem_o1)

    for b in range(_B):
        pltpu.sync_copy(rank_hbm.at[b], rank_v)

        @plsc.parallel_loop(0, _N // _L, unroll=4)
        def _(j):
            rv = rank_v[pl.ds(j * _L, _L)]
            iv = j * _L + lanes
            plsc.store_scatter(idx_v, [rv], iv, mask=rv < _K)

        pltpu.sync_copy(idx_v.at[pl.ds(base, _RPW)],
                        idx_hbm.at[b].at[pl.ds(base, _RPW)])

        pltpu.async_copy(xg_hbm.at[b].at[idx_v.at[pl.ds(base, _RPW)]],
                         xrows_v, sem_x).wait()
        pltpu.sync_copy(xrows_v, newx_hbm.at[b].at[pl.ds(base, _RPW)])

        nblk = _RPW // _RB
        descs = [None, None]
        odescs = [None, None]
        descs[0] = pltpu.async_copy(
            a_hbm.at[b].at[idx_v.at[pl.ds(base, _RB)]], bufs[0], sems[0])
        for t in range(nblk):
            if t + 1 < nblk:
                descs[(t + 1) % 2] = pltpu.async_copy(
                    a_hbm.at[b].at[idx_v.at[pl.ds(base + (t + 1) * _RB, _RB)]],
                    bufs[(t + 1) % 2], sems[(t + 1) % 2])
            descs[t % 2].wait()
            if odescs[t % 2] is not None:
                odescs[t % 2].wait()
            rows = bufs[t % 2]
            aout = outs[t % 2]

            @plsc.parallel_loop(0, _K // _L, unroll=4)
            def _(j, rows=rows, aout=aout):
                cv = idx_v[pl.ds(j * _L, _L)]
                for r in range(_RB):
                    rv = jnp.full((_L,), r, jnp.int32)
                    aout[r, pl.ds(j * _L, _L)] = plsc.load_gather(
                        rows, [rv, cv])

            odescs[t % 2] = pltpu.async_copy(
                aout, newa_hbm.at[b].at[pl.ds(base + t * _RB, _RB)],
                osems[t % 2])
        for dsc in odescs:
            if dsc is not None:
                dsc.wait()


@functools.cache
def _sc_gather():
    return pl.kernel(
        _sc_body,
        out_type=[
            jax.ShapeDtypeStruct((_B, _K, _F), jnp.float32),
            jax.ShapeDtypeStruct((_B, _K, _K), jnp.float32),
            jax.ShapeDtypeStruct((_B, _K), jnp.int32),
        ],
        mesh=plsc.VectorSubcoreMesh(core_axis_name="c", subcore_axis_name="s"),
        compiler_params=pltpu.CompilerParams(needs_layout_passes=False),
        scratch_types=[
            pltpu.VMEM((_N,), jnp.int32),
            pltpu.VMEM((_K,), jnp.int32),
            pltpu.VMEM((_RPW, _F), jnp.float32),
            pltpu.VMEM((_RB, _N), jnp.float32),
            pltpu.VMEM((_RB, _N), jnp.float32),
            pltpu.VMEM((_RB, _K), jnp.float32),
            pltpu.VMEM((_RB, _K), jnp.float32),
            pltpu.SemaphoreType.DMA,
            pltpu.SemaphoreType.DMA,
            pltpu.SemaphoreType.DMA,
            pltpu.SemaphoreType.DMA,
            pltpu.SemaphoreType.DMA,
        ],
    )


def kernel(X, A, kernel, training):
    y3, xg = pl.pallas_call(
        _scores_body,
        grid=(_B, _N // _IT_A),
        in_specs=[
            pl.BlockSpec((1, _IT_A, _F), lambda b, i: (b, i, 0)),
            pl.BlockSpec((_F, 1), lambda b, i: (0, 0)),
        ],
        out_specs=[
            pl.BlockSpec((1, _IT_A, 1), lambda b, i: (b, i, 0)),
            pl.BlockSpec((1, _IT_A, _F), lambda b, i: (b, i, 0)),
        ],
        out_shape=[
            jax.ShapeDtypeStruct((_B, _N, 1), jnp.float32),
            jax.ShapeDtypeStruct((_B, _N, _F), jnp.float32),
        ],
    )(X, kernel)

    y_row = y3.reshape(_B, 1, _N)
    rank3 = pl.pallas_call(
        _rank_body,
        grid=(_B, _N // _IT_R),
        in_specs=[
            pl.BlockSpec((1, _IT_R, 1), lambda b, i: (b, i, 0)),
            pl.BlockSpec((1, 1, _N), lambda b, i: (b, 0, 0)),
        ],
        out_specs=pl.BlockSpec((1, _IT_R, 1), lambda b, i: (b, i, 0)),
        out_shape=jax.ShapeDtypeStruct((_B, _N, 1), jnp.int32),
    )(y3, y_row)

    rank = rank3.reshape(_B, _N)
    new_x, new_a, idx = _sc_gather()(xg, A, rank)
    return (new_x, new_a, idx[..., None])


# speedup vs baseline: 2.0778x; 1.0141x over previous
import functools

import jax
import jax.numpy as jnp
from jax import lax
from jax.experimental import pallas as pl
from jax.experimental.pallas import tpu as pltpu
from jax.experimental.pallas import tpu_sc as plsc

_B, _N, _F = 4, 4096, 128
_K = 2048
_IT_A = 2048
_IT_R = 1024
_NW = 32
_RPW = _K // _NW
_RB = 8
_L = 16


def _scores_body(x_ref, kn_ref, y_ref, xg_ref):
    kn = kn_ref[...]
    sq = jnp.sum(kn * kn)
    knn = kn * lax.rsqrt(jnp.maximum(sq, 1e-12))
    x = x_ref[0]
    y = lax.dot_general(
        x.astype(jnp.bfloat16), knn.astype(jnp.bfloat16),
        (((1,), (0,)), ((), ())),
        preferred_element_type=jnp.float32)
    y_ref[0] = y
    xg_ref[0] = x * jnp.tanh(y)


def _rank_body(yi_ref, yj_ref, r_ref):
    yi = yi_ref[0]
    yj = yj_ref[0]
    i0 = pl.program_id(1) * _IT_R
    ii = i0 + lax.broadcasted_iota(jnp.int32, (_IT_R, _N), 0)
    jj = lax.broadcasted_iota(jnp.int32, (_IT_R, _N), 1)
    one, zero = jnp.float32(1), jnp.float32(0)
    ge = jnp.where(yj >= yi, one, zero)
    gt = jnp.where(yj > yi, one, zero)
    cnt = jnp.where(jj < ii, ge, gt)
    ones = jnp.ones((_N, 1), jnp.float32)
    s = lax.dot_general(cnt, ones, (((1,), (0,)), ((), ())),
                        preferred_element_type=jnp.float32)
    r_ref[0] = s.astype(jnp.int32)


def _sc_body(xg_hbm, a_hbm, rank_hbm, newx_hbm, newa_hbm, idx_hbm,
             rank_v, idx_v, xrows_v, arows0_v, arows1_v, aout0_v, aout1_v,
             sem_x, sem_a0, sem_a1, sem_o0, sem_o1):
    wid = lax.axis_index("s") * 2 + lax.axis_index("c")
    base = wid * _RPW
    lanes = lax.iota(jnp.int32, _L)
    bufs = (arows0_v, arows1_v)
    sems = (sem_a0, sem_a1)
    outs = (aout0_v, aout1_v)
    osems = (sem_o0, sem_o1)

    for b in range(_B):
        pltpu.sync_copy(rank_hbm.at[b], rank_v)

        @plsc.parallel_loop(0, _N // _L, unroll=4)
        def _(j):
            rv = rank_v[pl.ds(j * _L, _L)]
            iv = j * _L + lanes
            plsc.store_scatter(idx_v, [rv], iv, mask=rv < _K)

        pltpu.sync_copy(idx_v.at[pl.ds(base, _RPW)],
                        idx_hbm.at[b].at[pl.ds(base, _RPW)])

        pltpu.async_copy(xg_hbm.at[b].at[idx_v.at[pl.ds(base, _RPW)]],
                         xrows_v, sem_x).wait()
        pltpu.sync_copy(xrows_v, newx_hbm.at[b].at[pl.ds(base, _RPW)])

        nblk = _RPW // _RB
        descs = [None, None]
        odescs = [None, None]
        descs[0] = pltpu.async_copy(
            a_hbm.at[b].at[idx_v.at[pl.ds(base, _RB)]], bufs[0], sems[0])
        for t in range(nblk):
            if t + 1 < nblk:
                descs[(t + 1) % 2] = pltpu.async_copy(
                    a_hbm.at[b].at[idx_v.at[pl.ds(base + (t + 1) * _RB, _RB)]],
                    bufs[(t + 1) % 2], sems[(t + 1) % 2])
            descs[t % 2].wait()
            if odescs[t % 2] is not None:
                odescs[t % 2].wait()
            rows = bufs[t % 2]
            aout = outs[t % 2]

            @plsc.parallel_loop(0, _K // _L, unroll=4)
            def _(j, rows=rows, aout=aout):
                cv = idx_v[pl.ds(j * _L, _L)]
                for r in range(_RB):
                    rv = jnp.full((_L,), r, jnp.int32)
                    aout[r, pl.ds(j * _L, _L)] = plsc.load_gather(
                        rows, [rv, cv])

            odescs[t % 2] = pltpu.async_copy(
                aout, newa_hbm.at[b].at[pl.ds(base + t * _RB, _RB)],
                osems[t % 2])
        for dsc in odescs:
            if dsc is not None:
                dsc.wait()


@functools.cache
def _sc_gather():
    return pl.kernel(
        _sc_body,
        out_type=[
            jax.ShapeDtypeStruct((_B, _K, _F), jnp.float32),
            jax.ShapeDtypeStruct((_B, _K, _K), jnp.float32),
            jax.ShapeDtypeStruct((_B, _K), jnp.int32),
        ],
        mesh=plsc.VectorSubcoreMesh(core_axis_name="c", subcore_axis_name="s"),
        compiler_params=pltpu.CompilerParams(needs_layout_passes=False),
        scratch_types=[
            pltpu.VMEM((_N,), jnp.int32),
            pltpu.VMEM((_K,), jnp.int32),
            pltpu.VMEM((_RPW, _F), jnp.float32),
            pltpu.VMEM((_RB, _N), jnp.float32),
            pltpu.VMEM((_RB, _N), jnp.float32),
            pltpu.VMEM((_RB, _K), jnp.float32),
            pltpu.VMEM((_RB, _K), jnp.float32),
            pltpu.SemaphoreType.DMA,
            pltpu.SemaphoreType.DMA,
            pltpu.SemaphoreType.DMA,
            pltpu.SemaphoreType.DMA,
            pltpu.SemaphoreType.DMA,
        ],
    )


def kernel(X, A, kernel, training):
    y3, xg = pl.pallas_call(
        _scores_body,
        grid=(_B, _N // _IT_A),
        in_specs=[
            pl.BlockSpec((1, _IT_A, _F), lambda b, i: (b, i, 0)),
            pl.BlockSpec((_F, 1), lambda b, i: (0, 0)),
        ],
        out_specs=[
            pl.BlockSpec((1, _IT_A, 1), lambda b, i: (b, i, 0)),
            pl.BlockSpec((1, _IT_A, _F), lambda b, i: (b, i, 0)),
        ],
        out_shape=[
            jax.ShapeDtypeStruct((_B, _N, 1), jnp.float32),
            jax.ShapeDtypeStruct((_B, _N, _F), jnp.float32),
        ],
    )(X, kernel)

    y_row = y3.reshape(_B, 1, _N)
    rank3 = pl.pallas_call(
        _rank_body,
        grid=(_B, _N // _IT_R),
        in_specs=[
            pl.BlockSpec((1, _IT_R, 1), lambda b, i: (b, i, 0)),
            pl.BlockSpec((1, 1, _N), lambda b, i: (b, 0, 0)),
        ],
        out_specs=pl.BlockSpec((1, _IT_R, 1), lambda b, i: (b, i, 0)),
        out_shape=jax.ShapeDtypeStruct((_B, _N, 1), jnp.int32),
    )(y3, y_row)

    rank = rank3.reshape(_B, _N)
    new_x, new_a, idx = _sc_gather()(xg, A, rank)
    return (new_x, new_a, idx[..., None])
